# Initial kernel scaffold; baseline (speedup 1.0000x reference)
import jax
import jax.numpy as jnp
from jax.experimental import pallas as pl


def kernel(state, edge_index, W1, b1, W2, b2):
    raise NotImplementedError("write your pallas kernel here")


# speedup vs baseline: 104.0555x; 104.0555x over previous
import functools

import jax
import jax.numpy as jnp
from jax import lax
from jax.experimental import pallas as pl
from jax.experimental.pallas import tpu as pltpu
from jax.experimental.pallas import tpu_sc as plsc

_L = 16


def _make_sc_kernels(n_nodes, n_edges, nc, ns):
    nw = nc * ns
    ept = n_edges // nw
    bat = 128
    nch = -(-ept // bat)
    epad = nch * bat
    acc_n = -(-(n_nodes + 1) // (ns * _L)) * (ns * _L)
    stripe = acc_n // ns

    mesh = plsc.VectorSubcoreMesh(core_axis_name="c", subcore_axis_name="s")
    out_t = jax.ShapeDtypeStruct((nc, acc_n), jnp.float32)
    cparams = pltpu.CompilerParams(needs_layout_passes=False)

    @functools.partial(
        pl.kernel,
        out_type=out_t,
        mesh=mesh,
        scratch_types=[
            pltpu.VMEM((nch, bat), jnp.int32),
            pltpu.VMEM((bat,), jnp.float32),
            pltpu.VMEM((stripe,), jnp.float32),
            pltpu.VMEM_SHARED((acc_n,), jnp.float32),
        ],
        compiler_params=cparams,
    )
    def deg_kernel(dst_hbm, out_hbm, dst_v, ones_v, z_v, acc_sh):
        c = lax.axis_index("c")
        s = lax.axis_index("s")
        wid = s * nc + c

        def fill(ref, count, val):
            def body(k, _):
                ref[pl.ds(k * _L, _L)] = jnp.full((_L,), val, jnp.float32)
                return 0
            lax.fori_loop(0, count, body, 0)

        fill(z_v, stripe // _L, 0.0)
        fill(ones_v, bat // _L, 1.0)
        pltpu.sync_copy(z_v, acc_sh.at[pl.ds(s * stripe, stripe)])
        pltpu.sync_copy(dst_hbm.at[wid], dst_v)
        plsc.subcore_barrier()

        def scat(j, _):
            pltpu.sync_copy(ones_v, acc_sh.at[dst_v.at[j]], add=True)
            return 0
        lax.fori_loop(0, nch, scat, 0)
        plsc.subcore_barrier()
        pltpu.sync_copy(acc_sh.at[pl.ds(s * stripe, stripe)],
                        out_hbm.at[c, pl.ds(s * stripe, stripe)])

    @functools.partial(
        pl.kernel,
        out_type=out_t,
        mesh=mesh,
        scratch_types=[
            pltpu.VMEM((epad,), jnp.int32),
            pltpu.VMEM((nch, bat), jnp.int32),
            pltpu.VMEM((epad,), jnp.float32),
            pltpu.VMEM((n_nodes,), jnp.float32),
            pltpu.VMEM((stripe,), jnp.float32),
            pltpu.VMEM_SHARED((acc_n,), jnp.float32),
        ],
        compiler_params=cparams,
    )
    def edge_kernel(src_hbm, dst_hbm, t_hbm, out_hbm,
                    src_v, dst_v, vals_v, t_v, z_v, acc_sh):
        c = lax.axis_index("c")
        s = lax.axis_index("s")
        wid = s * nc + c

        def zf(k, _):
            z_v[pl.ds(k * _L, _L)] = jnp.zeros((_L,), jnp.float32)
            return 0
        lax.fori_loop(0, stripe // _L, zf, 0)
        pltpu.sync_copy(z_v, acc_sh.at[pl.ds(s * stripe, stripe)])
        pltpu.sync_copy(src_hbm.at[wid], src_v)
        pltpu.sync_copy(dst_hbm.at[wid], dst_v)
        pltpu.sync_copy(t_hbm, t_v)

        def gath(k, _):
            iv = src_v[pl.ds(k * _L, _L)]
            vals_v[pl.ds(k * _L, _L)] = plsc.load_gather(t_v, [iv])
            return 0
        lax.fori_loop(0, epad // _L, gath, 0)
        plsc.subcore_barrier()

        def scat(j, _):
            pltpu.sync_copy(vals_v.at[pl.ds(j * bat, bat)],
                            acc_sh.at[dst_v.at[j]], add=True)
            return 0
        lax.fori_loop(0, nch, scat, 0)
        plsc.subcore_barrier()
        pltpu.sync_copy(acc_sh.at[pl.ds(s * stripe, stripe)],
                        out_hbm.at[c, pl.ds(s * stripe, stripe)])

    return deg_kernel, edge_kernel, epad, acc_n


def _t_body(n_nodes, state_ref, w1_ref, w2_ref, cnt_ref, t_ref, dinv_ref):
    w = jnp.dot(w1_ref[...], w2_ref[...],
                preferred_element_type=jnp.float32)[:, 0]
    s = jnp.sum(state_ref[...] * w[None, :], axis=1)
    cnt = cnt_ref[0, :n_nodes] + cnt_ref[1, :n_nodes]
    dinv = lax.rsqrt(cnt + 1.0)
    t_ref[...] = dinv * s
    dinv_ref[...] = dinv


def _final_body(n_nodes, acc_ref, t_ref, dinv_ref, b1_ref, w2_ref, b2_ref,
                out_ref):
    acc = acc_ref[0, :n_nodes] + acc_ref[1, :n_nodes]
    cconst = jnp.sum(b1_ref[...] * w2_ref[...]) + jnp.sum(b2_ref[...])
    out_ref[...] = dinv_ref[...] * (acc + t_ref[...]) + cconst


def kernel(state, edge_index, W1, b1, W2, b2):
    n_nodes, d_in = state.shape
    n_edges = edge_index.shape[1]
    info = plsc.get_sparse_core_info()
    nc, ns = info.num_cores, info.num_subcores
    nw = nc * ns

    deg_k, edge_k, epad, acc_n = _make_sc_kernels(n_nodes, n_edges, nc, ns)
    ept = n_edges // nw

    src = edge_index[0].reshape(nw, ept)
    dst = edge_index[1].reshape(nw, ept)
    pad = epad - ept
    src_p = jnp.pad(src, ((0, 0), (0, pad)))
    dst_p = jnp.pad(dst, ((0, 0), (0, pad)), constant_values=n_nodes)
    dst_r = dst_p.reshape(nw, epad // 128, 128)

    cnt2 = deg_k(dst_r)

    t, dinv = pl.pallas_call(
        functools.partial(_t_body, n_nodes),
        out_shape=[jax.ShapeDtypeStruct((n_nodes,), jnp.float32)] * 2,
    )(state, W1, W2, cnt2)

    acc2 = edge_k(src_p, dst_r, t)

    out = pl.pallas_call(
        functools.partial(_final_body, n_nodes),
        out_shape=jax.ShapeDtypeStruct((n_nodes,), jnp.float32),
    )(acc2, t, dinv, b1, W2[:, 0], b2)
    return out.reshape(n_nodes, 1)


# speedup vs baseline: 121.4384x; 1.1671x over previous
import functools

import jax
import jax.numpy as jnp
from jax import lax
from jax.experimental import pallas as pl
from jax.experimental.pallas import tpu as pltpu
from jax.experimental.pallas import tpu_sc as plsc

_L = 16


def _make_sc_kernels(n_nodes, n_edges, nc, ns):
    nw = nc * ns
    ept = n_edges // nw
    nvec = ept // _L

    mesh = plsc.VectorSubcoreMesh(core_axis_name="c", subcore_axis_name="s")
    out_t = jax.ShapeDtypeStruct((nw, n_nodes), jnp.float32)
    cparams = pltpu.CompilerParams(needs_layout_passes=False)

    @functools.partial(
        pl.kernel,
        out_type=out_t,
        mesh=mesh,
        scratch_types=[
            pltpu.VMEM((ept,), jnp.int32),
            pltpu.VMEM((n_nodes,), jnp.float32),
        ],
        compiler_params=cparams,
    )
    def deg_kernel(dst_hbm, out_hbm, dst_v, acc_v):
        c = lax.axis_index("c")
        s = lax.axis_index("s")
        wid = s * nc + c

        def zf(k, _):
            acc_v[pl.ds(k * _L, _L)] = jnp.zeros((_L,), jnp.float32)
            return 0
        lax.fori_loop(0, n_nodes // _L, zf, 0)
        pltpu.sync_copy(dst_hbm.at[pl.ds(wid * ept, ept)], dst_v)
        ones = jnp.ones((_L,), jnp.float32)

        def scat(k, _):
            iv = dst_v[pl.ds(k * _L, _L)]
            plsc.addupdate_scatter(acc_v, [iv], ones)
            return 0
        lax.fori_loop(0, nvec, scat, 0)
        pltpu.sync_copy(acc_v, out_hbm.at[wid])

    @functools.partial(
        pl.kernel,
        out_type=out_t,
        mesh=mesh,
        scratch_types=[
            pltpu.VMEM((ept,), jnp.int32),
            pltpu.VMEM((ept,), jnp.int32),
            pltpu.VMEM((n_nodes,), jnp.float32),
            pltpu.VMEM((n_nodes,), jnp.float32),
        ],
        compiler_params=cparams,
    )
    def edge_kernel(src_hbm, dst_hbm, t_hbm, out_hbm, src_v, dst_v, t_v, acc_v):
        c = lax.axis_index("c")
        s = lax.axis_index("s")
        wid = s * nc + c

        def zf(k, _):
            acc_v[pl.ds(k * _L, _L)] = jnp.zeros((_L,), jnp.float32)
            return 0
        lax.fori_loop(0, n_nodes // _L, zf, 0)
        pltpu.sync_copy(src_hbm.at[pl.ds(wid * ept, ept)], src_v)
        pltpu.sync_copy(dst_hbm.at[pl.ds(wid * ept, ept)], dst_v)
        pltpu.sync_copy(t_hbm, t_v)

        def body(k, _):
            sv = src_v[pl.ds(k * _L, _L)]
            dv = dst_v[pl.ds(k * _L, _L)]
            vals = plsc.load_gather(t_v, [sv])
            plsc.addupdate_scatter(acc_v, [dv], vals)
            return 0
        lax.fori_loop(0, nvec, body, 0)
        pltpu.sync_copy(acc_v, out_hbm.at[wid])

    return deg_kernel, edge_kernel


def _t_body(state_ref, w1_ref, w2_ref, cnt_ref, t_ref, dinv_ref):
    w_row = lax.dot_general(w2_ref[...], w1_ref[...],
                            (((0,), (1,)), ((), ())),
                            preferred_element_type=jnp.float32)
    s_row = lax.dot_general(w_row, state_ref[...],
                            (((1,), (1,)), ((), ())),
                            preferred_element_type=jnp.float32)
    cnt = jnp.sum(cnt_ref[...], axis=0)
    dinv = lax.rsqrt(cnt + 1.0)
    t_ref[...] = dinv * s_row[0]
    dinv_ref[...] = dinv


def _final_body(acc_ref, t_ref, dinv_ref, b1_ref, w2_ref, b2_ref, out_ref):
    acc = jnp.sum(acc_ref[...], axis=0)
    cconst = jnp.sum(b1_ref[...] * w2_ref[...]) + jnp.sum(b2_ref[...])
    out_ref[...] = dinv_ref[...] * (acc + t_ref[...]) + cconst


def kernel(state, edge_index, W1, b1, W2, b2):
    n_nodes, _ = state.shape
    n_edges = edge_index.shape[1]
    info = plsc.get_sparse_core_info()
    nc, ns = info.num_cores, info.num_subcores

    deg_k, edge_k = _make_sc_kernels(n_nodes, n_edges, nc, ns)

    src = edge_index[0]
    dst = edge_index[1]
    cnt32 = deg_k(dst)

    t, dinv = pl.pallas_call(
        _t_body,
        out_shape=[jax.ShapeDtypeStruct((n_nodes,), jnp.float32)] * 2,
    )(state, W1, W2, cnt32)

    acc32 = edge_k(src, dst, t)

    out = pl.pallas_call(
        _final_body,
        out_shape=jax.ShapeDtypeStruct((n_nodes,), jnp.float32),
    )(acc32, t, dinv, b1, W2[:, 0], b2)
    return out.reshape(n_nodes, 1)


# speedup vs baseline: 182.5740x; 1.5034x over previous
import functools

import jax
import jax.numpy as jnp
from jax import lax
from jax.experimental import pallas as pl
from jax.experimental.pallas import tpu as pltpu
from jax.experimental.pallas import tpu_sc as plsc

_L = 16


def _make_sc_kernels(n_nodes, n_edges, nc, ns):
    nw = nc * ns
    nchunk = n_edges // 128
    wch = -(-nchunk // nw) + 1
    zvec = n_nodes // _L

    mesh = plsc.VectorSubcoreMesh(core_axis_name="c", subcore_axis_name="s")
    out_t = jax.ShapeDtypeStruct((nw, n_nodes), jnp.float32)
    cparams = pltpu.CompilerParams(needs_layout_passes=False)

    def chunk_bounds(wid):
        start = (nchunk * wid) // nw
        end = (nchunk * (wid + 1)) // nw
        base = jnp.minimum(start, nchunk - wch)
        return start, end, base

    def zero_acc(acc_v):
        zero = jnp.zeros((_L,), jnp.float32)

        def zf(k, _):
            for j in range(8):
                acc_v[pl.ds((k * 8 + j) * _L, _L)] = zero
            return 0
        lax.fori_loop(0, zvec // 8, zf, 0)
        for j in range(zvec - (zvec // 8) * 8):
            acc_v[pl.ds(((zvec // 8) * 8 + j) * _L, _L)] = zero

    @functools.partial(
        pl.kernel,
        out_type=out_t,
        mesh=mesh,
        scratch_types=[
            pltpu.VMEM((2, wch * 128), jnp.int32),
            pltpu.VMEM((n_nodes,), jnp.float32),
            pltpu.SemaphoreType.DMA,
        ],
        compiler_params=cparams,
    )
    def deg_kernel(edge_hbm, out_hbm, ed_v, acc_v, sem):
        c = lax.axis_index("c")
        s = lax.axis_index("s")
        wid = s * nc + c
        start, end, base = chunk_bounds(wid)
        cp = pltpu.async_copy(
            edge_hbm.at[:, pl.ds(base * 128, wch * 128)], ed_v, sem)
        zero_acc(acc_v)
        cp.wait()
        ones = jnp.ones((_L,), jnp.float32)
        off0 = (start - base) * 128

        @plsc.parallel_loop(0, (end - start) * 128, 128, unroll=1)
        def scat(k):
            for j in range(8):
                iv = ed_v[1, pl.ds(off0 + k + j * _L, _L)]
                plsc.addupdate_scatter(acc_v, [iv], ones)
        pltpu.sync_copy(acc_v, out_hbm.at[wid])

    @functools.partial(
        pl.kernel,
        out_type=out_t,
        mesh=mesh,
        scratch_types=[
            pltpu.VMEM((2, wch * 128), jnp.int32),
            pltpu.VMEM((n_nodes,), jnp.float32),
            pltpu.VMEM((n_nodes,), jnp.float32),
            pltpu.SemaphoreType.DMA,
        ],
        compiler_params=cparams,
    )
    def edge_kernel(edge_hbm, t_hbm, out_hbm, ed_v, t_v, acc_v, sem):
        c = lax.axis_index("c")
        s = lax.axis_index("s")
        wid = s * nc + c
        start, end, base = chunk_bounds(wid)
        cp1 = pltpu.async_copy(
            edge_hbm.at[:, pl.ds(base * 128, wch * 128)], ed_v, sem)
        cp2 = pltpu.async_copy(t_hbm, t_v, sem)
        zero_acc(acc_v)
        cp1.wait()
        cp2.wait()
        off0 = (start - base) * 128

        @plsc.parallel_loop(0, (end - start) * 128, 128, unroll=1)
        def body(k):
            for j in range(8):
                sv = ed_v[0, pl.ds(off0 + k + j * _L, _L)]
                dv = ed_v[1, pl.ds(off0 + k + j * _L, _L)]
                vals = plsc.load_gather(t_v, [sv])
                plsc.addupdate_scatter(acc_v, [dv], vals)
        pltpu.sync_copy(acc_v, out_hbm.at[wid])

    return deg_kernel, edge_kernel


def _s_body(state_ref, w1_ref, w2_ref, s_ref):
    w_row = lax.dot_general(w2_ref[...], w1_ref[...],
                            (((0,), (1,)), ((), ())),
                            preferred_element_type=jnp.float32)
    s_row = lax.dot_general(w_row, state_ref[...],
                            (((1,), (1,)), ((), ())),
                            preferred_element_type=jnp.float32)
    s_ref[...] = s_row[0]


def _t_body(s_ref, cnt_ref, t_ref, dinv_ref):
    cnt = jnp.sum(cnt_ref[...], axis=0)
    dinv = lax.rsqrt(cnt + 1.0)
    t_ref[...] = dinv * s_ref[...]
    dinv_ref[...] = dinv


def _final_body(acc_ref, t_ref, dinv_ref, b1_ref, w2_ref, b2_ref, out_ref):
    acc = jnp.sum(acc_ref[...], axis=0)
    cconst = jnp.sum(b1_ref[...] * w2_ref[...]) + jnp.sum(b2_ref[...])
    val = dinv_ref[...] * (acc + t_ref[...]) + cconst
    out_ref[...] = val[:, None]


def kernel(state, edge_index, W1, b1, W2, b2):
    n_nodes, _ = state.shape
    n_edges = edge_index.shape[1]
    info = plsc.get_sparse_core_info()
    nc, ns = info.num_cores, info.num_subcores

    deg_k, edge_k = _make_sc_kernels(n_nodes, n_edges, nc, ns)

    cnt32 = deg_k(edge_index)

    s = pl.pallas_call(
        _s_body,
        out_shape=jax.ShapeDtypeStruct((n_nodes,), jnp.float32),
    )(state, W1, W2)

    t, dinv = pl.pallas_call(
        _t_body,
        out_shape=[jax.ShapeDtypeStruct((n_nodes,), jnp.float32)] * 2,
    )(s, cnt32)

    acc32 = edge_k(edge_index, t)

    out = pl.pallas_call(
        _final_body,
        out_shape=jax.ShapeDtypeStruct((n_nodes, 1), jnp.float32),
    )(acc32, t, dinv, b1, W2[:, 0], b2)
    return out


# speedup vs baseline: 209.9451x; 1.1499x over previous
import functools

import jax
import jax.numpy as jnp
from jax import lax
from jax.experimental import pallas as pl
from jax.experimental.pallas import tpu as pltpu
from jax.experimental.pallas import tpu_sc as plsc

_L = 16


def _make_sc_kernels(n_nodes, n_edges, nc, ns):
    nw = nc * ns
    nchunk = n_edges // 128
    wch = -(-nchunk // nw) + 1
    zvec = n_nodes // _L

    mesh = plsc.VectorSubcoreMesh(core_axis_name="c", subcore_axis_name="s")
    out_t = jax.ShapeDtypeStruct((nw, n_nodes), jnp.float32)
    cparams = pltpu.CompilerParams(needs_layout_passes=False)

    def chunk_bounds(wid):
        start = (nchunk * wid) // nw
        end = (nchunk * (wid + 1)) // nw
        base = jnp.minimum(start, nchunk - wch)
        return start, end, base

    def zero_acc(acc_v):
        zero = jnp.zeros((_L,), jnp.float32)

        def zf(k, _):
            for j in range(8):
                acc_v[pl.ds((k * 8 + j) * _L, _L)] = zero
            return 0
        lax.fori_loop(0, zvec // 8, zf, 0)
        for j in range(zvec - (zvec // 8) * 8):
            acc_v[pl.ds(((zvec // 8) * 8 + j) * _L, _L)] = zero

    @functools.partial(
        pl.kernel,
        out_type=out_t,
        mesh=mesh,
        scratch_types=[
            pltpu.VMEM((2, wch * 128), jnp.int32),
            pltpu.VMEM((n_nodes,), jnp.float32),
            pltpu.SemaphoreType.DMA,
        ],
        compiler_params=cparams,
    )
    def deg_kernel(edge_hbm, out_hbm, ed_v, acc_v, sem):
        c = lax.axis_index("c")
        s = lax.axis_index("s")
        wid = s * nc + c
        start, end, base = chunk_bounds(wid)
        cp = pltpu.async_copy(
            edge_hbm.at[:, pl.ds(base * 128, wch * 128)], ed_v, sem)
        zero_acc(acc_v)
        cp.wait()
        ones = jnp.ones((_L,), jnp.float32)
        off0 = (start - base) * 128

        @plsc.parallel_loop(0, (end - start) * 128, 128, unroll=1)
        def scat(k):
            for j in range(8):
                iv = ed_v[1, pl.ds(off0 + k + j * _L, _L)]
                plsc.addupdate_scatter(acc_v, [iv], ones)
        pltpu.sync_copy(acc_v, out_hbm.at[wid])

    @functools.partial(
        pl.kernel,
        out_type=out_t,
        mesh=mesh,
        scratch_types=[
            pltpu.VMEM((2, wch * 128), jnp.int32),
            pltpu.VMEM((n_nodes,), jnp.float32),
            pltpu.VMEM((n_nodes,), jnp.float32),
            pltpu.SemaphoreType.DMA,
        ],
        compiler_params=cparams,
    )
    def edge_kernel(edge_hbm, t_hbm, out_hbm, ed_v, t_v, acc_v, sem):
        c = lax.axis_index("c")
        s = lax.axis_index("s")
        wid = s * nc + c
        start, end, base = chunk_bounds(wid)
        cp1 = pltpu.async_copy(
            edge_hbm.at[:, pl.ds(base * 128, wch * 128)], ed_v, sem)
        cp2 = pltpu.async_copy(t_hbm, t_v, sem)
        zero_acc(acc_v)
        cp1.wait()
        cp2.wait()
        off0 = (start - base) * 128

        @plsc.parallel_loop(0, (end - start) * 128, 128, unroll=1)
        def body(k):
            for j in range(8):
                sv = ed_v[0, pl.ds(off0 + k + j * _L, _L)]
                dv = ed_v[1, pl.ds(off0 + k + j * _L, _L)]
                vals = plsc.load_gather(t_v, [sv])
                plsc.addupdate_scatter(acc_v, [dv], vals)
        pltpu.sync_copy(acc_v, out_hbm.at[wid])

    return deg_kernel, edge_kernel


def _s_body(state_ref, w1_ref, w2_ref, s_ref):
    w_row = lax.dot_general(w2_ref[...], w1_ref[...],
                            (((0,), (1,)), ((), ())),
                            preferred_element_type=jnp.float32)
    s_row = lax.dot_general(w_row, state_ref[...],
                            (((1,), (1,)), ((), ())),
                            preferred_element_type=jnp.float32)
    s_ref[...] = s_row[0]


def _t_body(s_ref, cnt_ref, t_ref, dinv_ref):
    cnt = jnp.sum(cnt_ref[...], axis=0)
    dinv = lax.rsqrt(cnt + 1.0)
    t_ref[...] = dinv * s_ref[...]
    dinv_ref[...] = dinv


def _final_body(acc_ref, t_ref, dinv_ref, b1_ref, w2_ref, b2_ref, out_ref):
    acc = jnp.sum(acc_ref[...], axis=0)
    cconst = jnp.sum(b1_ref[...] * w2_ref[...]) + jnp.sum(b2_ref[...])
    out_ref[...] = dinv_ref[...] * (acc + t_ref[...]) + cconst


def kernel(state, edge_index, W1, b1, W2, b2):
    n_nodes, _ = state.shape
    n_edges = edge_index.shape[1]
    info = plsc.get_sparse_core_info()
    nc, ns = info.num_cores, info.num_subcores

    deg_k, edge_k = _make_sc_kernels(n_nodes, n_edges, nc, ns)

    cnt32 = deg_k(edge_index)

    s = pl.pallas_call(
        _s_body,
        out_shape=jax.ShapeDtypeStruct((n_nodes,), jnp.float32),
    )(state, W1, W2)

    t, dinv = pl.pallas_call(
        _t_body,
        out_shape=[jax.ShapeDtypeStruct((n_nodes,), jnp.float32)] * 2,
    )(s, cnt32)

    acc32 = edge_k(edge_index, t)

    out = pl.pallas_call(
        _final_body,
        out_shape=jax.ShapeDtypeStruct((n_nodes,), jnp.float32),
    )(acc32, t, dinv, b1, W2[:, 0], b2)
    return out.reshape(n_nodes, 1)


# speedup vs baseline: 211.0998x; 1.0055x over previous
import functools

import jax
import jax.numpy as jnp
from jax import lax
from jax.experimental import pallas as pl
from jax.experimental.pallas import tpu as pltpu
from jax.experimental.pallas import tpu_sc as plsc

_L = 16


def _make_sc_kernels(n_nodes, n_edges, nc, ns):
    nw = nc * ns
    nchunk = n_edges // 128
    wch = -(-nchunk // nw) + 1
    zvec = n_nodes // _L

    mesh = plsc.VectorSubcoreMesh(core_axis_name="c", subcore_axis_name="s")
    out_t = jax.ShapeDtypeStruct((nw, n_nodes), jnp.float32)
    cparams = pltpu.CompilerParams(needs_layout_passes=False)

    def chunk_bounds(wid):
        start = (nchunk * wid) // nw
        end = (nchunk * (wid + 1)) // nw
        base = jnp.minimum(start, nchunk - wch)
        return start, end, base

    def zero_acc(acc_v):
        zero = jnp.zeros((_L,), jnp.float32)

        def zf(k, _):
            for j in range(8):
                acc_v[pl.ds((k * 8 + j) * _L, _L)] = zero
            return 0
        lax.fori_loop(0, zvec // 8, zf, 0)
        for j in range(zvec - (zvec // 8) * 8):
            acc_v[pl.ds(((zvec // 8) * 8 + j) * _L, _L)] = zero

    @functools.partial(
        pl.kernel,
        out_type=out_t,
        mesh=mesh,
        scratch_types=[
            pltpu.VMEM((2, wch * 128), jnp.int32),
            pltpu.VMEM((n_nodes,), jnp.float32),
            pltpu.SemaphoreType.DMA,
        ],
        compiler_params=cparams,
    )
    def deg_kernel(edge_hbm, out_hbm, ed_v, acc_v, sem):
        c = lax.axis_index("c")
        s = lax.axis_index("s")
        wid = s * nc + c
        start, end, base = chunk_bounds(wid)
        cp = pltpu.async_copy(
            edge_hbm.at[:, pl.ds(base * 128, wch * 128)], ed_v, sem)
        zero_acc(acc_v)
        cp.wait()
        ones = jnp.ones((_L,), jnp.float32)
        off0 = (start - base) * 128

        @plsc.parallel_loop(0, (end - start) * 128, _L, unroll=8)
        def scat(k):
            iv = ed_v[1, pl.ds(off0 + k, _L)]
            plsc.addupdate_scatter(acc_v, [iv], ones)
        pltpu.sync_copy(acc_v, out_hbm.at[wid])

    @functools.partial(
        pl.kernel,
        out_type=out_t,
        mesh=mesh,
        scratch_types=[
            pltpu.VMEM((2, wch * 128), jnp.int32),
            pltpu.VMEM((n_nodes,), jnp.float32),
            pltpu.VMEM((n_nodes,), jnp.float32),
            pltpu.SemaphoreType.DMA,
        ],
        compiler_params=cparams,
    )
    def edge_kernel(edge_hbm, t_hbm, out_hbm, ed_v, t_v, acc_v, sem):
        c = lax.axis_index("c")
        s = lax.axis_index("s")
        wid = s * nc + c
        start, end, base = chunk_bounds(wid)
        cp1 = pltpu.async_copy(
            edge_hbm.at[:, pl.ds(base * 128, wch * 128)], ed_v, sem)
        cp2 = pltpu.async_copy(t_hbm, t_v, sem)
        zero_acc(acc_v)
        cp1.wait()
        cp2.wait()
        off0 = (start - base) * 128

        @plsc.parallel_loop(0, (end - start) * 128, _L, unroll=8)
        def body(k):
            sv = ed_v[0, pl.ds(off0 + k, _L)]
            dv = ed_v[1, pl.ds(off0 + k, _L)]
            vals = plsc.load_gather(t_v, [sv])
            plsc.addupdate_scatter(acc_v, [dv], vals)
        pltpu.sync_copy(acc_v, out_hbm.at[wid])

    return deg_kernel, edge_kernel


def _s_body(state_ref, w1_ref, w2_ref, s_ref):
    w_row = lax.dot_general(w2_ref[...], w1_ref[...],
                            (((0,), (1,)), ((), ())),
                            preferred_element_type=jnp.float32)
    s_row = lax.dot_general(w_row, state_ref[...],
                            (((1,), (1,)), ((), ())),
                            preferred_element_type=jnp.float32)
    s_ref[...] = s_row[0]


def _t_body(s_ref, cnt_ref, t_ref, dinv_ref):
    cnt = jnp.sum(cnt_ref[...], axis=0)
    dinv = lax.rsqrt(cnt + 1.0)
    t_ref[...] = dinv * s_ref[...]
    dinv_ref[...] = dinv


def _final_body(acc_ref, t_ref, dinv_ref, b1_ref, w2_ref, b2_ref, out_ref):
    acc = jnp.sum(acc_ref[...], axis=0)
    cconst = jnp.sum(b1_ref[...] * w2_ref[...]) + jnp.sum(b2_ref[...])
    out_ref[...] = dinv_ref[...] * (acc + t_ref[...]) + cconst


def kernel(state, edge_index, W1, b1, W2, b2):
    n_nodes, _ = state.shape
    n_edges = edge_index.shape[1]
    info = plsc.get_sparse_core_info()
    nc, ns = info.num_cores, info.num_subcores

    deg_k, edge_k = _make_sc_kernels(n_nodes, n_edges, nc, ns)

    cnt32 = deg_k(edge_index)

    s = pl.pallas_call(
        _s_body,
        out_shape=jax.ShapeDtypeStruct((n_nodes,), jnp.float32),
    )(state, W1, W2)

    t, dinv = pl.pallas_call(
        _t_body,
        out_shape=[jax.ShapeDtypeStruct((n_nodes,), jnp.float32)] * 2,
    )(s, cnt32)

    acc32 = edge_k(edge_index, t)

    out = pl.pallas_call(
        _final_body,
        out_shape=jax.ShapeDtypeStruct((n_nodes,), jnp.float32),
    )(acc32, t, dinv, b1, W2[:, 0], b2)
    return out.reshape(n_nodes, 1)
